# baseline (speedup 1.0000x reference)
import functools

import jax
import jax.numpy as jnp
from jax import lax
from jax.experimental import pallas as pl
from jax.experimental.pallas import tpu as pltpu
from jax.experimental.pallas import tpu_sc as plsc

_NUM_CORES = 2
_NUM_SUBCORES = 16
_NW = _NUM_CORES * _NUM_SUBCORES


_VB = 16384


def _tr_body(x_ref, o_ref):
  x = x_ref[...]
  xc = jnp.concatenate([x[:, :_VB], x[:, _VB:]], axis=0)
  o_ref[...] = xc.T


def _transpose_detile(table):
  D, V = table.shape
  grid = -(-V // (2 * _VB))
  out = pl.pallas_call(
      _tr_body,
      grid=(grid,),
      in_specs=[pl.BlockSpec((D, 2 * _VB), lambda i: (0, i))],
      out_specs=pl.BlockSpec((_VB, 2 * D), lambda i: (i, 0)),
      out_shape=jax.ShapeDtypeStruct((grid * _VB, 2 * D), jnp.float32),
  )(table)
  return out.reshape(2 * grid * _VB, D)


def _make_gather_pool(B, CTX, D, b_per_w):
  mesh = plsc.VectorSubcoreMesh(
      core_axis_name="c", subcore_axis_name="s", num_cores=_NUM_CORES,
      num_subcores=_NUM_SUBCORES)

  @functools.partial(
      pl.kernel,
      mesh=mesh,
      compiler_params=pltpu.CompilerParams(use_tc_tiling_on_sc=False),
      out_type=jax.ShapeDtypeStruct((B, D), jnp.float32),
      scratch_types=[
          pltpu.VMEM((CTX, b_per_w), jnp.int32),
          pltpu.VMEM((b_per_w, D), jnp.float32),
          pltpu.SemaphoreType.DMA,
      ],
  )
  def gather_pool(idx_hbm, table_hbm, out_hbm, idx_t, acc_v, sem):
    wid = lax.axis_index("s") * _NUM_CORES + lax.axis_index("c")
    base = wid * b_per_w
    pltpu.sync_copy(idx_hbm.at[wid], idx_t)

    pltpu.async_copy(table_hbm.at[idx_t.at[0]], acc_v, sem).wait()

    def fire(j, carry):
      pltpu.async_copy(table_hbm.at[idx_t.at[j]], acc_v, sem, add=True)
      return carry

    lax.fori_loop(1, CTX, fire, 0)

    def drain(j, carry):
      pltpu.make_async_copy(table_hbm.at[idx_t.at[0]], acc_v, sem).wait()
      return carry

    lax.fori_loop(1, CTX, drain, 0)

    pltpu.sync_copy(acc_v, out_hbm.at[pl.ds(base, b_per_w)])

  return gather_pool


def _linear_body(w_ref, x_ref, b_ref, o_ref):
  o_ref[...] = (
      lax.dot_general(
          w_ref[...], x_ref[...], (((1,), (1,)), ((), ())),
          preferred_element_type=jnp.float32)
      + b_ref[...]
  )


def _linear(pooled, w, bias_col, bm):
  B, D = pooled.shape
  N = w.shape[0]
  return pl.pallas_call(
      _linear_body,
      grid=(B // bm,),
      in_specs=[
          pl.BlockSpec((N, D), lambda i: (0, 0)),
          pl.BlockSpec((bm, D), lambda i: (i, 0)),
          pl.BlockSpec((N, 1), lambda i: (0, 0)),
      ],
      out_specs=pl.BlockSpec((N, bm), lambda i: (0, i)),
      out_shape=jax.ShapeDtypeStruct((N, B), jnp.float32),
  )(w, pooled, bias_col)


def kernel(inputs, embed_table, fc_w, fc_b):
  B, CTX = inputs.shape
  V, D = embed_table.shape
  N = fc_w.shape[0]
  b_per_w = B // _NW

  v = inputs.astype(jnp.int32)
  idx = 2 * ((v // (2 * _VB)) * _VB + (v % _VB)) + ((v // _VB) % 2)
  idx = jnp.transpose(idx.reshape(_NW, b_per_w, CTX), (0, 2, 1))
  table_lin = _transpose_detile(jnp.transpose(embed_table))
  pooled = _make_gather_pool(B, CTX, D, b_per_w)(idx, table_lin)
  logits_t = _linear(pooled, fc_w, fc_b.reshape(N, 1), bm=512)
  return jnp.transpose(logits_t)
